# baseline (speedup 1.0000x reference)
import functools

import jax
import jax.numpy as jnp
from jax import lax
from jax.experimental import pallas as pl
from jax.experimental.pallas import tpu as pltpu
from jax.experimental.pallas import tpu_sc as plsc

F = 128
N_CELL = 50000
N_GENE = 50000
E = 200000

NC = 2
NS = 16
NW = NC * NS
L = 16

PASSES = 3
NBUCK = PASSES * NW
ROWS = 528
NPAD = NBUCK * ROWS
CH = 128
CHB = CH + 16
EP = 200448

_BLK = 1000


def _splat_i32(x):
    return jnp.zeros((L,), jnp.int32) + x


def _sread(ref, i):
    v = plsc.load_gather(ref, [_splat_i32(i)])
    return jnp.max(v)


def _seg_body(is_max, x_h, src_h, w_h, dst_h, starts_h, out_h,
              acc, sb0, sb1, wb0, wb1, db0, db1, il0, il1, we0, we1,
              dl0, dl1, rw0, rw1, st_v, sem0, sem1):
    c = lax.axis_index("c")
    s = lax.axis_index("s")
    wid = s * NC + c
    iot = lax.iota(jnp.int32, L)
    bufs = ((sb0, wb0, db0, il0, we0, dl0, rw0, sem0),
            (sb1, wb1, db1, il1, we1, dl1, rw1, sem1))
    pltpu.sync_copy(starts_h, st_v)
    for p in range(PASSES):
        g = p * NW + wid

        def zero_body(r, _):
            rs = _splat_i32(r)
            for k in range(F // L):
                plsc.store_scatter(acc, [rs, iot + k * L],
                                   jnp.zeros((L,), jnp.float32))
            return 0
        lax.fori_loop(0, ROWS, zero_body, 0)

        lo = _sread(st_v, g)
        hi = _sread(st_v, g + 1)
        abase = (lo // 8) * 8
        n_raw = (hi - abase + CH - 1) // CH
        n_pairs = (n_raw + 1) // 2

        def stage(cb, b):
            sb, wb, db, il, we, dl, rw, sem = bufs[b]
            pltpu.sync_copy(src_h.at[pl.ds(cb, CHB)], sb)
            pltpu.sync_copy(w_h.at[pl.ds(cb, CHB)], wb)
            pltpu.sync_copy(dst_h.at[pl.ds(cb, CHB)], db)
            for m in range(CH // L):
                sl = pl.ds(m * L, L)
                eidx = cb + m * L + iot
                valid = (eidx >= lo) & (eidx < hi)
                il[sl] = jnp.where(valid, sb[sl], 0)
                we[sl] = jnp.where(valid, wb[sl], jnp.float32(0.0))
                dl[sl] = jnp.clip(db[sl] - g * ROWS, 0, ROWS - 1)
            pltpu.async_copy(x_h.at[il], rw, sem)

        def drain(b):
            sb, wb, db, il, we, dl, rw, sem = bufs[b]
            pltpu.make_async_copy(x_h.at[il], rw, sem).wait()
            U = 8

            def edge_body(eg, _):
                base = eg * U
                es = [_splat_i32(base + j) for j in range(U)]
                ws = [plsc.load_gather(we, [e]) for e in es]
                dsv = [plsc.load_gather(dl, [e]) for e in es]
                for k in range(F // L):
                    col = iot + k * L
                    for j in range(U):
                        rv = plsc.load_gather(rw, [es[j], col]) * ws[j]
                        if is_max:
                            cur = plsc.load_gather(acc, [dsv[j], col])
                            plsc.store_scatter(acc, [dsv[j], col],
                                               jnp.maximum(cur, rv))
                        else:
                            plsc.addupdate_scatter(acc, [dsv[j], col], rv)
                return 0
            lax.fori_loop(0, CH // U, edge_body, 0)

        stage(abase, 0)

        def pair_body(ip, _):
            cb = abase + (2 * ip) * CH
            stage(cb + CH, 1)
            drain(0)
            stage(cb + 2 * CH, 0)
            drain(1)
            return 0
        lax.fori_loop(0, n_pairs, pair_body, 0)
        pltpu.make_async_copy(x_h.at[il0], rw0, sem0).wait()
        pltpu.sync_copy(acc, out_h.at[pl.ds(g * ROWS, ROWS)])


def _make_seg(is_max):
    mesh = plsc.VectorSubcoreMesh(core_axis_name="c", subcore_axis_name="s",
                                  num_cores=NC, num_subcores=NS)
    return pl.kernel(
        functools.partial(_seg_body, is_max),
        out_type=jax.ShapeDtypeStruct((NPAD, F), jnp.float32),
        mesh=mesh,
        compiler_params=pltpu.CompilerParams(needs_layout_passes=False),
        scratch_types=[
            pltpu.VMEM((ROWS, F), jnp.float32),
            pltpu.VMEM((CHB,), jnp.int32),
            pltpu.VMEM((CHB,), jnp.int32),
            pltpu.VMEM((CHB,), jnp.float32),
            pltpu.VMEM((CHB,), jnp.float32),
            pltpu.VMEM((CHB,), jnp.int32),
            pltpu.VMEM((CHB,), jnp.int32),
            pltpu.VMEM((CH,), jnp.int32),
            pltpu.VMEM((CH,), jnp.int32),
            pltpu.VMEM((CH,), jnp.float32),
            pltpu.VMEM((CH,), jnp.float32),
            pltpu.VMEM((CH,), jnp.int32),
            pltpu.VMEM((CH,), jnp.int32),
            pltpu.VMEM((CH, F), jnp.float32),
            pltpu.VMEM((CH, F), jnp.float32),
            pltpu.VMEM((104,), jnp.int32),
            pltpu.SemaphoreType.DMA,
            pltpu.SemaphoreType.DMA,
        ],
    )


def _deg_body(dst_h, starts_h, out_h, dacc, dst_b, dval, dstl, st_v):
    c = lax.axis_index("c")
    s = lax.axis_index("s")
    wid = s * NC + c
    iot = lax.iota(jnp.int32, L)
    lane0 = iot == 0
    pltpu.sync_copy(starts_h, st_v)
    for p in range(PASSES):
        g = p * NW + wid

        for r in range(8):
            for k in range(F // L):
                plsc.store_scatter(dacc, [_splat_i32(r), iot + k * L],
                                   jnp.zeros((L,), jnp.float32))

        lo = _sread(st_v, g)
        hi = _sread(st_v, g + 1)
        abase = (lo // 8) * 8
        n_chunks = (hi - abase + CH - 1) // CH

        def chunk_body(i, _):
            cb = abase + i * CH
            pltpu.sync_copy(dst_h.at[pl.ds(cb, CHB)], dst_b)
            for m in range(CH // L):
                sl = pl.ds(m * L, L)
                eidx = cb + m * L + iot
                valid = (eidx >= lo) & (eidx < hi)
                dval[sl] = jnp.where(valid, jnp.float32(1.0), jnp.float32(0.0))
                dstl[sl] = jnp.clip(dst_b[sl] - g * ROWS, 0, ROWS - 1)

            def edge_body(e, _):
                es = _splat_i32(e)
                vspl = plsc.load_gather(dval, [es])
                dspl = plsc.load_gather(dstl, [es])
                plsc.addupdate_scatter(dacc, [jnp.right_shift(dspl, 7),
                                              dspl & 127], vspl, mask=lane0)
                return 0
            lax.fori_loop(0, CH, edge_body, 0)
            return 0
        lax.fori_loop(0, n_chunks, chunk_body, 0)
        pltpu.sync_copy(dacc, out_h.at[g])


def _make_deg():
    mesh = plsc.VectorSubcoreMesh(core_axis_name="c", subcore_axis_name="s",
                                  num_cores=NC, num_subcores=NS)
    return pl.kernel(
        _deg_body,
        out_type=jax.ShapeDtypeStruct((NBUCK, 8, F), jnp.float32),
        mesh=mesh,
        compiler_params=pltpu.CompilerParams(needs_layout_passes=False),
        scratch_types=[
            pltpu.VMEM((8, F), jnp.float32),
            pltpu.VMEM((CHB,), jnp.int32),
            pltpu.VMEM((CH,), jnp.float32),
            pltpu.VMEM((CH,), jnp.int32),
            pltpu.VMEM((104,), jnp.int32),
        ],
    )


GR = 1568
GCH = 224
GPAD = NW * GR


def _gather_body(tab_h, idx_h, out_h, idx_b, rows, sem):
    c = lax.axis_index("c")
    s = lax.axis_index("s")
    wid = s * NC + c
    base = wid * GR
    for i in range(GR // GCH):
        off = base + i * GCH
        pltpu.sync_copy(idx_h.at[pl.ds(off, GCH)], idx_b)
        pltpu.async_copy(tab_h.at[idx_b], rows, sem).wait()
        pltpu.sync_copy(rows, out_h.at[pl.ds(off, GCH)])


def _make_gather():
    mesh = plsc.VectorSubcoreMesh(core_axis_name="c", subcore_axis_name="s",
                                  num_cores=NC, num_subcores=NS)
    return pl.kernel(
        _gather_body,
        out_type=jax.ShapeDtypeStruct((GPAD, F), jnp.float32),
        mesh=mesh,
        compiler_params=pltpu.CompilerParams(needs_layout_passes=False),
        scratch_types=[
            pltpu.VMEM((GCH,), jnp.int32),
            pltpu.VMEM((GCH, F), jnp.float32),
            pltpu.SemaphoreType.DMA,
        ],
    )


_seg_sum_k = _make_seg(False)
_seg_max_k = _make_seg(True)
_deg_k = _make_deg()
_gather_k = _make_gather()


def _prep_rel(edge, w):
    src, dst = edge[0], edge[1]
    order = jnp.argsort(dst)
    ssrc = jnp.pad(src[order], (0, EP - E))
    sw = jnp.pad(w[order], (0, EP - E))
    sdst = jnp.pad(dst[order], (0, EP - E))
    bounds = jnp.arange(NBUCK + 1, dtype=jnp.int32) * ROWS
    starts = jnp.searchsorted(dst[order], bounds, side='left').astype(jnp.int32)
    starts = jnp.pad(starts, (0, 104 - (NBUCK + 1)))
    return ssrc, sw, sdst, starts


def _seg_sum(x, rel):
    ssrc, sw, sdst, starts = rel
    return _seg_sum_k(x, ssrc, sw, sdst, starts)[:N_GENE]


def _seg_max(x, rel):
    ssrc, sw, sdst, starts = rel
    return _seg_max_k(x, ssrc, sw, sdst, starts)[:N_GENE]


def _degree(rel):
    _, _, sdst, starts = rel
    d = _deg_k(sdst, starts).reshape(NBUCK, 8 * F)[:, :ROWS]
    return d.reshape(NPAD)[:N_GENE]


def _fused_linear(xs, ws, scales, bias, residual, do_relu):
    n = xs[0].shape[0]
    nx = len(xs)
    grid = n // _BLK
    modes = [None if sc is None else sc[1] for sc in scales]
    args = []
    in_specs = []
    for x in xs:
        args.append(x)
        in_specs.append(pl.BlockSpec((_BLK, F), lambda i: (i, 0)))
    for w in ws:
        args.append(w)
        in_specs.append(pl.BlockSpec((F, F), lambda i: (0, 0)))
    for sc in scales:
        if sc is not None:
            args.append(sc[0].reshape(n, 1))
            in_specs.append(pl.BlockSpec((_BLK, 1), lambda i: (i, 0)))
    args.append(bias.reshape(1, F))
    in_specs.append(pl.BlockSpec((1, F), lambda i: (0, 0)))
    if residual is not None:
        args.append(residual)
        in_specs.append(pl.BlockSpec((_BLK, F), lambda i: (i, 0)))

    def body(*refs):
        it = iter(refs)
        xrs = [next(it) for _ in range(nx)]
        wrs = [next(it) for _ in range(nx)]
        srs = [next(it) if m is not None else None for m in modes]
        brf = next(it)
        rrf = next(it) if residual is not None else None
        orf = next(it)
        acc = None
        for xr, wr, sr, mode in zip(xrs, wrs, srs, modes):
            xv = xr[...]
            if mode == 'mean':
                xv = xv / jnp.maximum(sr[...], 1.0)
            elif mode == 'gcn':
                xv = xv / (sr[...] + 1.0)
            t = jnp.dot(xv, wr[...], preferred_element_type=jnp.float32)
            acc = t if acc is None else acc + t
        acc = acc + brf[...]
        if do_relu:
            acc = jnp.maximum(acc, 0.0)
        if rrf is not None:
            acc = rrf[...] + acc
        orf[...] = acc

    return pl.pallas_call(
        body,
        grid=(grid,),
        in_specs=in_specs,
        out_specs=pl.BlockSpec((_BLK, F), lambda i: (i, 0)),
        out_shape=jax.ShapeDtypeStruct((n, F), jnp.float32),
    )(*args)


def kernel(cell_feat, gene_idx, edge_c2g, w_c2g, edge_g2c, w_g2c, edge_g2g,
           w_g2g, params):
    p = params

    rel_c2g = _prep_rel(edge_c2g, w_c2g)
    rel_g2c = _prep_rel(edge_g2c, w_g2c)
    rel_g2g = _prep_rel(edge_g2g, w_g2g)

    deg = {'c2g': _degree(rel_c2g),
           'g2c': _degree(rel_g2c),
           'g2g': _degree(rel_g2g)}
    rels = {'c2g': rel_c2g, 'g2c': rel_g2c, 'g2g': rel_g2g}

    def conv_mean(cp, h_cell, h_gene, res_cell, res_gene):
        s1 = _seg_sum(h_cell, rels['c2g'])
        s2 = _seg_sum(h_gene, rels['g2g'])
        out_gene = _fused_linear(
            [h_gene, s1, h_gene, s2],
            [cp['c2g']['W_self'], cp['c2g']['W_neigh'],
             cp['g2g']['W_self'], cp['g2g']['W_neigh']],
            [None, (deg['c2g'], 'mean'), None, (deg['g2g'], 'mean')],
            cp['c2g']['b'] + cp['g2g']['b'], res_gene, True)
        s3 = _seg_sum(h_gene, rels['g2c'])
        out_cell = _fused_linear(
            [h_cell, s3],
            [cp['g2c']['W_self'], cp['g2c']['W_neigh']],
            [None, (deg['g2c'], 'mean')],
            cp['g2c']['b'], res_cell, True)
        return out_cell, out_gene

    def conv_gcn(cp, h_cell, h_gene, res_cell, res_gene):
        s1 = _seg_sum(h_cell, rels['c2g'])
        s2 = _seg_sum(h_gene, rels['g2g'])
        out_gene = _fused_linear(
            [s1, h_gene, s2, h_gene],
            [cp['c2g']['W_neigh'], cp['c2g']['W_neigh'],
             cp['g2g']['W_neigh'], cp['g2g']['W_neigh']],
            [(deg['c2g'], 'gcn'), (deg['c2g'], 'gcn'),
             (deg['g2g'], 'gcn'), (deg['g2g'], 'gcn')],
            cp['c2g']['b'] + cp['g2g']['b'], res_gene, True)
        s3 = _seg_sum(h_gene, rels['g2c'])
        out_cell = _fused_linear(
            [s3, h_cell],
            [cp['g2c']['W_neigh'], cp['g2c']['W_neigh']],
            [(deg['g2c'], 'gcn'), (deg['g2c'], 'gcn')],
            cp['g2c']['b'], res_cell, True)
        return out_cell, out_gene

    def conv_pool(cp, h_cell, h_gene, res_cell, res_gene):
        pre_c2g = _fused_linear([h_cell], [cp['c2g']['W_pool']], [None],
                                cp['c2g']['b_pool'], None, True)
        pre_g2g = _fused_linear([h_gene], [cp['g2g']['W_pool']], [None],
                                cp['g2g']['b_pool'], None, True)
        pre_g2c = _fused_linear([h_gene], [cp['g2c']['W_pool']], [None],
                                cp['g2c']['b_pool'], None, True)
        m1 = _seg_max(pre_c2g, rels['c2g'])
        m2 = _seg_max(pre_g2g, rels['g2g'])
        out_gene = _fused_linear(
            [h_gene, m1, h_gene, m2],
            [cp['c2g']['W_self'], cp['c2g']['W_neigh'],
             cp['g2g']['W_self'], cp['g2g']['W_neigh']],
            [None, None, None, None],
            cp['c2g']['b'] + cp['g2g']['b'], res_gene, True)
        m3 = _seg_max(pre_g2c, rels['g2c'])
        out_cell = _fused_linear(
            [h_cell, m3],
            [cp['g2c']['W_self'], cp['g2c']['W_neigh']],
            [None, None],
            cp['g2c']['b'], res_cell, True)
        return out_cell, out_gene

    gidx = jnp.pad(gene_idx, (0, GPAD - N_GENE)).astype(jnp.int32)
    h_gene = _gather_k(p['embed'], gidx)[:N_GENE]
    h_cell = cell_feat

    h_cell, h_gene = conv_mean(p['conv1'], h_cell, h_gene, None, None)
    h_cell, h_gene = conv_gcn(p['conv2'], h_cell, h_gene, h_cell, h_gene)
    h_cell, h_gene = conv_pool(p['conv3'], h_cell, h_gene, h_cell, h_gene)
    h_cell = _fused_linear([h_cell], [p['lin1']['cell']['W']], [None],
                           p['lin1']['cell']['b'], None, False)
    h_gene = _fused_linear([h_gene], [p['lin1']['gene']['W']], [None],
                           p['lin1']['gene']['b'], None, False)
    h_cell, h_gene = conv_mean(p['conv4'], h_cell, h_gene, h_cell, h_gene)
    h_cell, h_gene = conv_gcn(p['conv5'], h_cell, h_gene, h_cell, h_gene)
    h_cell, h_gene = conv_pool(p['conv6'], h_cell, h_gene, h_cell, h_gene)
    h_cell = _fused_linear([h_cell], [p['lin2']['cell']['W']], [None],
                           p['lin2']['cell']['b'], None, False)
    h_gene = _fused_linear([h_gene], [p['lin2']['gene']['W']], [None],
                           p['lin2']['gene']['b'], None, False)
    return jnp.concatenate([h_cell, h_gene], axis=0)


# speedup vs baseline: 1.4396x; 1.4396x over previous
import functools

import jax
import jax.numpy as jnp
from jax import lax
from jax.experimental import pallas as pl
from jax.experimental.pallas import tpu as pltpu
from jax.experimental.pallas import tpu_sc as plsc

F = 128
N_CELL = 50000
N_GENE = 50000
E = 200000

NC = 2
NS = 16
NW = NC * NS
L = 16

PASSES = 3
NBUCK = PASSES * NW
ROWS = 528
NPAD = NBUCK * ROWS
CH = 256
CHB = CH + 16
EP = 200960

_BLK = 1000


def _splat_i32(x):
    return jnp.zeros((L,), jnp.int32) + x


def _sread(ref, i):
    v = plsc.load_gather(ref, [_splat_i32(i)])
    return jnp.max(v)


def _seg_body(is_max, x_h, src_h, w_h, dst_h, starts_h, out_h,
              acc, src_b, w_b, dst_b, idxl, weff, dstl, rows, st_v, sem):
    c = lax.axis_index("c")
    s = lax.axis_index("s")
    wid = s * NC + c
    iot = lax.iota(jnp.int32, L)
    pltpu.sync_copy(starts_h, st_v)
    for p in range(PASSES):
        g = p * NW + wid

        def zero_body(r, _):
            rs = _splat_i32(r)
            for k in range(F // L):
                plsc.store_scatter(acc, [rs, iot + k * L],
                                   jnp.zeros((L,), jnp.float32))
            return 0
        lax.fori_loop(0, ROWS, zero_body, 0)

        lo = _sread(st_v, g)
        hi = _sread(st_v, g + 1)
        abase = (lo // 8) * 8
        n_chunks = (hi - abase + CH - 1) // CH

        def chunk_body(i, _):
            cb = abase + i * CH
            pltpu.sync_copy(src_h.at[pl.ds(cb, CHB)], src_b)
            pltpu.sync_copy(w_h.at[pl.ds(cb, CHB)], w_b)
            pltpu.sync_copy(dst_h.at[pl.ds(cb, CHB)], dst_b)
            for m in range(CH // L):
                sl = pl.ds(m * L, L)
                eidx = cb + m * L + iot
                valid = (eidx >= lo) & (eidx < hi)
                idxl[sl] = jnp.where(valid, src_b[sl], 0)
                weff[sl] = jnp.where(valid, w_b[sl], jnp.float32(0.0))
                dstl[sl] = jnp.clip(dst_b[sl] - g * ROWS, 0, ROWS - 1)
            pltpu.async_copy(x_h.at[idxl], rows, sem).wait()

            U = 8

            def edge_body(eg, _):
                base = eg * U
                es = [_splat_i32(base + j) for j in range(U)]
                ws = [plsc.load_gather(weff, [e]) for e in es]
                dsv = [plsc.load_gather(dstl, [e]) for e in es]
                for k in range(F // L):
                    col = iot + k * L
                    for j in range(U):
                        rv = plsc.load_gather(rows, [es[j], col]) * ws[j]
                        if is_max:
                            cur = plsc.load_gather(acc, [dsv[j], col])
                            plsc.store_scatter(acc, [dsv[j], col],
                                               jnp.maximum(cur, rv))
                        else:
                            plsc.addupdate_scatter(acc, [dsv[j], col], rv)
                return 0
            lax.fori_loop(0, CH // U, edge_body, 0)
            return 0
        lax.fori_loop(0, n_chunks, chunk_body, 0)
        pltpu.sync_copy(acc, out_h.at[pl.ds(g * ROWS, ROWS)])


def _make_seg(is_max):
    mesh = plsc.VectorSubcoreMesh(core_axis_name="c", subcore_axis_name="s",
                                  num_cores=NC, num_subcores=NS)
    return pl.kernel(
        functools.partial(_seg_body, is_max),
        out_type=jax.ShapeDtypeStruct((NPAD, F), jnp.float32),
        mesh=mesh,
        compiler_params=pltpu.CompilerParams(needs_layout_passes=False),
        scratch_types=[
            pltpu.VMEM((ROWS, F), jnp.float32),
            pltpu.VMEM((CHB,), jnp.int32),
            pltpu.VMEM((CHB,), jnp.float32),
            pltpu.VMEM((CHB,), jnp.int32),
            pltpu.VMEM((CH,), jnp.int32),
            pltpu.VMEM((CH,), jnp.float32),
            pltpu.VMEM((CH,), jnp.int32),
            pltpu.VMEM((CH, F), jnp.float32),
            pltpu.VMEM((104,), jnp.int32),
            pltpu.SemaphoreType.DMA,
        ],
    )


def _deg_body(dst_h, starts_h, out_h, dacc, dst_b, dval, dstl, st_v):
    c = lax.axis_index("c")
    s = lax.axis_index("s")
    wid = s * NC + c
    iot = lax.iota(jnp.int32, L)
    lane0 = iot == 0
    pltpu.sync_copy(starts_h, st_v)
    for p in range(PASSES):
        g = p * NW + wid

        for r in range(8):
            for k in range(F // L):
                plsc.store_scatter(dacc, [_splat_i32(r), iot + k * L],
                                   jnp.zeros((L,), jnp.float32))

        lo = _sread(st_v, g)
        hi = _sread(st_v, g + 1)
        abase = (lo // 8) * 8
        n_chunks = (hi - abase + CH - 1) // CH

        def chunk_body(i, _):
            cb = abase + i * CH
            pltpu.sync_copy(dst_h.at[pl.ds(cb, CHB)], dst_b)
            for m in range(CH // L):
                sl = pl.ds(m * L, L)
                eidx = cb + m * L + iot
                valid = (eidx >= lo) & (eidx < hi)
                dval[sl] = jnp.where(valid, jnp.float32(1.0), jnp.float32(0.0))
                dstl[sl] = jnp.clip(dst_b[sl] - g * ROWS, 0, ROWS - 1)

            def edge_body(e, _):
                es = _splat_i32(e)
                vspl = plsc.load_gather(dval, [es])
                dspl = plsc.load_gather(dstl, [es])
                plsc.addupdate_scatter(dacc, [jnp.right_shift(dspl, 7),
                                              dspl & 127], vspl, mask=lane0)
                return 0
            lax.fori_loop(0, CH, edge_body, 0)
            return 0
        lax.fori_loop(0, n_chunks, chunk_body, 0)
        pltpu.sync_copy(dacc, out_h.at[g])


def _make_deg():
    mesh = plsc.VectorSubcoreMesh(core_axis_name="c", subcore_axis_name="s",
                                  num_cores=NC, num_subcores=NS)
    return pl.kernel(
        _deg_body,
        out_type=jax.ShapeDtypeStruct((NBUCK, 8, F), jnp.float32),
        mesh=mesh,
        compiler_params=pltpu.CompilerParams(needs_layout_passes=False),
        scratch_types=[
            pltpu.VMEM((8, F), jnp.float32),
            pltpu.VMEM((CHB,), jnp.int32),
            pltpu.VMEM((CH,), jnp.float32),
            pltpu.VMEM((CH,), jnp.int32),
            pltpu.VMEM((104,), jnp.int32),
        ],
    )


GR = 1568
GCH = 224
GPAD = NW * GR


def _gather_body(tab_h, idx_h, out_h, idx_b, rows, sem):
    c = lax.axis_index("c")
    s = lax.axis_index("s")
    wid = s * NC + c
    base = wid * GR
    for i in range(GR // GCH):
        off = base + i * GCH
        pltpu.sync_copy(idx_h.at[pl.ds(off, GCH)], idx_b)
        pltpu.async_copy(tab_h.at[idx_b], rows, sem).wait()
        pltpu.sync_copy(rows, out_h.at[pl.ds(off, GCH)])


def _make_gather():
    mesh = plsc.VectorSubcoreMesh(core_axis_name="c", subcore_axis_name="s",
                                  num_cores=NC, num_subcores=NS)
    return pl.kernel(
        _gather_body,
        out_type=jax.ShapeDtypeStruct((GPAD, F), jnp.float32),
        mesh=mesh,
        compiler_params=pltpu.CompilerParams(needs_layout_passes=False),
        scratch_types=[
            pltpu.VMEM((GCH,), jnp.int32),
            pltpu.VMEM((GCH, F), jnp.float32),
            pltpu.SemaphoreType.DMA,
        ],
    )


_seg_sum_k = _make_seg(False)
_seg_max_k = _make_seg(True)
_deg_k = _make_deg()
_gather_k = _make_gather()


def _prep_rel(edge, w):
    src, dst = edge[0], edge[1]
    order = jnp.argsort(dst)
    ssrc = jnp.pad(src[order], (0, EP - E))
    sw = jnp.pad(w[order], (0, EP - E))
    sdst = jnp.pad(dst[order], (0, EP - E))
    bounds = jnp.arange(NBUCK + 1, dtype=jnp.int32) * ROWS
    starts = jnp.searchsorted(dst[order], bounds, side='left').astype(jnp.int32)
    starts = jnp.pad(starts, (0, 104 - (NBUCK + 1)))
    return ssrc, sw, sdst, starts


def _seg_sum(x, rel):
    ssrc, sw, sdst, starts = rel
    return _seg_sum_k(x, ssrc, sw, sdst, starts)[:N_GENE]


def _seg_max(x, rel):
    ssrc, sw, sdst, starts = rel
    return _seg_max_k(x, ssrc, sw, sdst, starts)[:N_GENE]


def _degree(rel):
    _, _, sdst, starts = rel
    d = _deg_k(sdst, starts).reshape(NBUCK, 8 * F)[:, :ROWS]
    return d.reshape(NPAD)[:N_GENE]


def _fused_linear(xs, ws, scales, bias, residual, do_relu):
    n = xs[0].shape[0]
    nx = len(xs)
    grid = n // _BLK
    modes = [None if sc is None else sc[1] for sc in scales]
    args = []
    in_specs = []
    for x in xs:
        args.append(x)
        in_specs.append(pl.BlockSpec((_BLK, F), lambda i: (i, 0)))
    for w in ws:
        args.append(w)
        in_specs.append(pl.BlockSpec((F, F), lambda i: (0, 0)))
    for sc in scales:
        if sc is not None:
            args.append(sc[0].reshape(n, 1))
            in_specs.append(pl.BlockSpec((_BLK, 1), lambda i: (i, 0)))
    args.append(bias.reshape(1, F))
    in_specs.append(pl.BlockSpec((1, F), lambda i: (0, 0)))
    if residual is not None:
        args.append(residual)
        in_specs.append(pl.BlockSpec((_BLK, F), lambda i: (i, 0)))

    def body(*refs):
        it = iter(refs)
        xrs = [next(it) for _ in range(nx)]
        wrs = [next(it) for _ in range(nx)]
        srs = [next(it) if m is not None else None for m in modes]
        brf = next(it)
        rrf = next(it) if residual is not None else None
        orf = next(it)
        acc = None
        for xr, wr, sr, mode in zip(xrs, wrs, srs, modes):
            xv = xr[...]
            if mode == 'mean':
                xv = xv / jnp.maximum(sr[...], 1.0)
            elif mode == 'gcn':
                xv = xv / (sr[...] + 1.0)
            t = jnp.dot(xv, wr[...], preferred_element_type=jnp.float32)
            acc = t if acc is None else acc + t
        acc = acc + brf[...]
        if do_relu:
            acc = jnp.maximum(acc, 0.0)
        if rrf is not None:
            acc = rrf[...] + acc
        orf[...] = acc

    return pl.pallas_call(
        body,
        grid=(grid,),
        in_specs=in_specs,
        out_specs=pl.BlockSpec((_BLK, F), lambda i: (i, 0)),
        out_shape=jax.ShapeDtypeStruct((n, F), jnp.float32),
    )(*args)


def kernel(cell_feat, gene_idx, edge_c2g, w_c2g, edge_g2c, w_g2c, edge_g2g,
           w_g2g, params):
    p = params

    rel_c2g = _prep_rel(edge_c2g, w_c2g)
    rel_g2c = _prep_rel(edge_g2c, w_g2c)
    rel_g2g = _prep_rel(edge_g2g, w_g2g)

    deg = {'c2g': _degree(rel_c2g),
           'g2c': _degree(rel_g2c),
           'g2g': _degree(rel_g2g)}
    rels = {'c2g': rel_c2g, 'g2c': rel_g2c, 'g2g': rel_g2g}

    def conv_mean(cp, h_cell, h_gene, res_cell, res_gene):
        s1 = _seg_sum(h_cell, rels['c2g'])
        s2 = _seg_sum(h_gene, rels['g2g'])
        out_gene = _fused_linear(
            [h_gene, s1, h_gene, s2],
            [cp['c2g']['W_self'], cp['c2g']['W_neigh'],
             cp['g2g']['W_self'], cp['g2g']['W_neigh']],
            [None, (deg['c2g'], 'mean'), None, (deg['g2g'], 'mean')],
            cp['c2g']['b'] + cp['g2g']['b'], res_gene, True)
        s3 = _seg_sum(h_gene, rels['g2c'])
        out_cell = _fused_linear(
            [h_cell, s3],
            [cp['g2c']['W_self'], cp['g2c']['W_neigh']],
            [None, (deg['g2c'], 'mean')],
            cp['g2c']['b'], res_cell, True)
        return out_cell, out_gene

    def conv_gcn(cp, h_cell, h_gene, res_cell, res_gene):
        s1 = _seg_sum(h_cell, rels['c2g'])
        s2 = _seg_sum(h_gene, rels['g2g'])
        out_gene = _fused_linear(
            [s1, h_gene, s2, h_gene],
            [cp['c2g']['W_neigh'], cp['c2g']['W_neigh'],
             cp['g2g']['W_neigh'], cp['g2g']['W_neigh']],
            [(deg['c2g'], 'gcn'), (deg['c2g'], 'gcn'),
             (deg['g2g'], 'gcn'), (deg['g2g'], 'gcn')],
            cp['c2g']['b'] + cp['g2g']['b'], res_gene, True)
        s3 = _seg_sum(h_gene, rels['g2c'])
        out_cell = _fused_linear(
            [s3, h_cell],
            [cp['g2c']['W_neigh'], cp['g2c']['W_neigh']],
            [(deg['g2c'], 'gcn'), (deg['g2c'], 'gcn')],
            cp['g2c']['b'], res_cell, True)
        return out_cell, out_gene

    def conv_pool(cp, h_cell, h_gene, res_cell, res_gene):
        pre_c2g = _fused_linear([h_cell], [cp['c2g']['W_pool']], [None],
                                cp['c2g']['b_pool'], None, True)
        pre_g2g = _fused_linear([h_gene], [cp['g2g']['W_pool']], [None],
                                cp['g2g']['b_pool'], None, True)
        pre_g2c = _fused_linear([h_gene], [cp['g2c']['W_pool']], [None],
                                cp['g2c']['b_pool'], None, True)
        m1 = _seg_max(pre_c2g, rels['c2g'])
        m2 = _seg_max(pre_g2g, rels['g2g'])
        out_gene = _fused_linear(
            [h_gene, m1, h_gene, m2],
            [cp['c2g']['W_self'], cp['c2g']['W_neigh'],
             cp['g2g']['W_self'], cp['g2g']['W_neigh']],
            [None, None, None, None],
            cp['c2g']['b'] + cp['g2g']['b'], res_gene, True)
        m3 = _seg_max(pre_g2c, rels['g2c'])
        out_cell = _fused_linear(
            [h_cell, m3],
            [cp['g2c']['W_self'], cp['g2c']['W_neigh']],
            [None, None],
            cp['g2c']['b'], res_cell, True)
        return out_cell, out_gene

    gidx = jnp.pad(gene_idx, (0, GPAD - N_GENE)).astype(jnp.int32)
    h_gene = _gather_k(p['embed'], gidx)[:N_GENE]
    h_cell = cell_feat

    h_cell, h_gene = conv_mean(p['conv1'], h_cell, h_gene, None, None)
    h_cell, h_gene = conv_gcn(p['conv2'], h_cell, h_gene, h_cell, h_gene)
    h_cell, h_gene = conv_pool(p['conv3'], h_cell, h_gene, h_cell, h_gene)
    h_cell = _fused_linear([h_cell], [p['lin1']['cell']['W']], [None],
                           p['lin1']['cell']['b'], None, False)
    h_gene = _fused_linear([h_gene], [p['lin1']['gene']['W']], [None],
                           p['lin1']['gene']['b'], None, False)
    h_cell, h_gene = conv_mean(p['conv4'], h_cell, h_gene, h_cell, h_gene)
    h_cell, h_gene = conv_gcn(p['conv5'], h_cell, h_gene, h_cell, h_gene)
    h_cell, h_gene = conv_pool(p['conv6'], h_cell, h_gene, h_cell, h_gene)
    h_cell = _fused_linear([h_cell], [p['lin2']['cell']['W']], [None],
                           p['lin2']['cell']['b'], None, False)
    h_gene = _fused_linear([h_gene], [p['lin2']['gene']['W']], [None],
                           p['lin2']['gene']['b'], None, False)
    return jnp.concatenate([h_cell, h_gene], axis=0)


# speedup vs baseline: 2.4068x; 1.6718x over previous
import functools

import jax
import jax.numpy as jnp
from jax import lax
from jax.experimental import pallas as pl
from jax.experimental.pallas import tpu as pltpu
from jax.experimental.pallas import tpu_sc as plsc

F = 128
N_CELL = 50000
N_GENE = 50000
E = 200000

NC = 2
NS = 16
NW = NC * NS
L = 16

PASSES = 3
NBUCK = PASSES * NW
ROWS = 528
NPAD = NBUCK * ROWS
CH = 128
CHB = CH + 16
EP = 200960

_BLK = 1000


def _splat_i32(x):
    return jnp.zeros((L,), jnp.int32) + x


def _sread(ref, i):
    v = plsc.load_gather(ref, [_splat_i32(i)])
    return jnp.max(v)


def _seg_body(is_max, x_h, src_h, w_h, dst_h, starts_h, out_h,
              acc, src_b, w_b, dst_b, idxl, weff, dstl, rows, st_v, sem):
    c = lax.axis_index("c")
    s = lax.axis_index("s")
    wid = s * NC + c
    iot = lax.iota(jnp.int32, L)
    pltpu.sync_copy(starts_h, st_v)
    for p in range(PASSES):
        g = p * NW + wid

        def zero_body(r, _):
            rs = _splat_i32(r)
            for k in range(F // L):
                plsc.store_scatter(acc, [rs, iot + k * L],
                                   jnp.zeros((L,), jnp.float32))
            return 0
        lax.fori_loop(0, ROWS, zero_body, 0)

        lo = _sread(st_v, g)
        hi = _sread(st_v, g + 1)
        abase = (lo // 8) * 8
        n_chunks = (hi - abase + CH - 1) // CH

        def chunk_body(i, _):
            cb = abase + i * CH
            pltpu.sync_copy(src_h.at[pl.ds(cb, CHB)], src_b)
            pltpu.sync_copy(w_h.at[pl.ds(cb, CHB)], w_b)
            pltpu.sync_copy(dst_h.at[pl.ds(cb, CHB)], dst_b)
            for m in range(CH // L):
                sl = pl.ds(m * L, L)
                eidx = cb + m * L + iot
                valid = (eidx >= lo) & (eidx < hi)
                idxl[sl] = jnp.where(valid, src_b[sl], 0)
                weff[sl] = jnp.where(valid, w_b[sl], jnp.float32(0.0))
                dstl[sl] = jnp.clip(dst_b[sl] - g * ROWS, 0, ROWS - 1)
            pltpu.async_copy(x_h.at[idxl], rows, sem).wait()

            if is_max:
                U = 8

                def edge_body(eg, _):
                    base = eg * U
                    es = [_splat_i32(base + j) for j in range(U)]
                    ws = [plsc.load_gather(weff, [e]) for e in es]
                    dsv = [plsc.load_gather(dstl, [e]) for e in es]
                    for k in range(F // L):
                        col = iot + k * L
                        for j in range(U):
                            rv = plsc.load_gather(rows, [es[j], col]) * ws[j]
                            cur = plsc.load_gather(acc, [dsv[j], col])
                            plsc.store_scatter(acc, [dsv[j], col],
                                               jnp.maximum(cur, rv))
                    return 0
                lax.fori_loop(0, CH // U, edge_body, 0)
            else:
                @plsc.parallel_loop(0, CH, step=1, unroll=4)
                def _(e):
                    es = _splat_i32(e)
                    ws = plsc.load_gather(weff, [es])
                    dsv = plsc.load_gather(dstl, [es])
                    for k in range(F // L):
                        col = iot + k * L
                        rv = plsc.load_gather(rows, [es, col]) * ws
                        plsc.addupdate_scatter(acc, [dsv, col], rv)
            return 0
        lax.fori_loop(0, n_chunks, chunk_body, 0)
        pltpu.sync_copy(acc, out_h.at[pl.ds(g * ROWS, ROWS)])


def _make_seg(is_max):
    mesh = plsc.VectorSubcoreMesh(core_axis_name="c", subcore_axis_name="s",
                                  num_cores=NC, num_subcores=NS)
    return pl.kernel(
        functools.partial(_seg_body, is_max),
        out_type=jax.ShapeDtypeStruct((NPAD, F), jnp.float32),
        mesh=mesh,
        compiler_params=pltpu.CompilerParams(needs_layout_passes=False),
        scratch_types=[
            pltpu.VMEM((ROWS, F), jnp.float32),
            pltpu.VMEM((CHB,), jnp.int32),
            pltpu.VMEM((CHB,), jnp.float32),
            pltpu.VMEM((CHB,), jnp.int32),
            pltpu.VMEM((CH,), jnp.int32),
            pltpu.VMEM((CH,), jnp.float32),
            pltpu.VMEM((CH,), jnp.int32),
            pltpu.VMEM((CH, F), jnp.float32),
            pltpu.VMEM((104,), jnp.int32),
            pltpu.SemaphoreType.DMA,
        ],
    )


def _deg_body(dst_h, starts_h, out_h, dacc, dst_b, dval, dstl, st_v):
    c = lax.axis_index("c")
    s = lax.axis_index("s")
    wid = s * NC + c
    iot = lax.iota(jnp.int32, L)
    lane0 = iot == 0
    pltpu.sync_copy(starts_h, st_v)
    for p in range(PASSES):
        g = p * NW + wid

        for r in range(8):
            for k in range(F // L):
                plsc.store_scatter(dacc, [_splat_i32(r), iot + k * L],
                                   jnp.zeros((L,), jnp.float32))

        lo = _sread(st_v, g)
        hi = _sread(st_v, g + 1)
        abase = (lo // 8) * 8
        n_chunks = (hi - abase + CH - 1) // CH

        def chunk_body(i, _):
            cb = abase + i * CH
            pltpu.sync_copy(dst_h.at[pl.ds(cb, CHB)], dst_b)
            for m in range(CH // L):
                sl = pl.ds(m * L, L)
                eidx = cb + m * L + iot
                valid = (eidx >= lo) & (eidx < hi)
                dval[sl] = jnp.where(valid, jnp.float32(1.0), jnp.float32(0.0))
                dstl[sl] = jnp.clip(dst_b[sl] - g * ROWS, 0, ROWS - 1)

            def edge_body(e, _):
                es = _splat_i32(e)
                vspl = plsc.load_gather(dval, [es])
                dspl = plsc.load_gather(dstl, [es])
                plsc.addupdate_scatter(dacc, [jnp.right_shift(dspl, 7),
                                              dspl & 127], vspl, mask=lane0)
                return 0
            lax.fori_loop(0, CH, edge_body, 0)
            return 0
        lax.fori_loop(0, n_chunks, chunk_body, 0)
        pltpu.sync_copy(dacc, out_h.at[g])


def _make_deg():
    mesh = plsc.VectorSubcoreMesh(core_axis_name="c", subcore_axis_name="s",
                                  num_cores=NC, num_subcores=NS)
    return pl.kernel(
        _deg_body,
        out_type=jax.ShapeDtypeStruct((NBUCK, 8, F), jnp.float32),
        mesh=mesh,
        compiler_params=pltpu.CompilerParams(needs_layout_passes=False),
        scratch_types=[
            pltpu.VMEM((8, F), jnp.float32),
            pltpu.VMEM((CHB,), jnp.int32),
            pltpu.VMEM((CH,), jnp.float32),
            pltpu.VMEM((CH,), jnp.int32),
            pltpu.VMEM((104,), jnp.int32),
        ],
    )


GR = 1568
GCH = 224
GPAD = NW * GR


def _gather_body(tab_h, idx_h, out_h, idx_b, rows, sem):
    c = lax.axis_index("c")
    s = lax.axis_index("s")
    wid = s * NC + c
    base = wid * GR
    for i in range(GR // GCH):
        off = base + i * GCH
        pltpu.sync_copy(idx_h.at[pl.ds(off, GCH)], idx_b)
        pltpu.async_copy(tab_h.at[idx_b], rows, sem).wait()
        pltpu.sync_copy(rows, out_h.at[pl.ds(off, GCH)])


def _make_gather():
    mesh = plsc.VectorSubcoreMesh(core_axis_name="c", subcore_axis_name="s",
                                  num_cores=NC, num_subcores=NS)
    return pl.kernel(
        _gather_body,
        out_type=jax.ShapeDtypeStruct((GPAD, F), jnp.float32),
        mesh=mesh,
        compiler_params=pltpu.CompilerParams(needs_layout_passes=False),
        scratch_types=[
            pltpu.VMEM((GCH,), jnp.int32),
            pltpu.VMEM((GCH, F), jnp.float32),
            pltpu.SemaphoreType.DMA,
        ],
    )


_seg_sum_k = _make_seg(False)
_seg_max_k = _make_seg(True)
_deg_k = _make_deg()
_gather_k = _make_gather()


def _prep_rel(edge, w):
    src, dst = edge[0], edge[1]
    order = jnp.argsort(dst)
    ssrc = jnp.pad(src[order], (0, EP - E))
    sw = jnp.pad(w[order], (0, EP - E))
    sdst = jnp.pad(dst[order], (0, EP - E))
    bounds = jnp.arange(NBUCK + 1, dtype=jnp.int32) * ROWS
    starts = jnp.searchsorted(dst[order], bounds, side='left').astype(jnp.int32)
    starts = jnp.pad(starts, (0, 104 - (NBUCK + 1)))
    return ssrc, sw, sdst, starts


def _seg_sum(x, rel):
    ssrc, sw, sdst, starts = rel
    return _seg_sum_k(x, ssrc, sw, sdst, starts)[:N_GENE]


def _seg_max(x, rel):
    ssrc, sw, sdst, starts = rel
    return _seg_max_k(x, ssrc, sw, sdst, starts)[:N_GENE]


def _degree(rel):
    _, _, sdst, starts = rel
    d = _deg_k(sdst, starts).reshape(NBUCK, 8 * F)[:, :ROWS]
    return d.reshape(NPAD)[:N_GENE]


def _fused_linear(xs, ws, scales, bias, residual, do_relu):
    n = xs[0].shape[0]
    nx = len(xs)
    grid = n // _BLK
    modes = [None if sc is None else sc[1] for sc in scales]
    args = []
    in_specs = []
    for x in xs:
        args.append(x)
        in_specs.append(pl.BlockSpec((_BLK, F), lambda i: (i, 0)))
    for w in ws:
        args.append(w)
        in_specs.append(pl.BlockSpec((F, F), lambda i: (0, 0)))
    for sc in scales:
        if sc is not None:
            args.append(sc[0].reshape(n, 1))
            in_specs.append(pl.BlockSpec((_BLK, 1), lambda i: (i, 0)))
    args.append(bias.reshape(1, F))
    in_specs.append(pl.BlockSpec((1, F), lambda i: (0, 0)))
    if residual is not None:
        args.append(residual)
        in_specs.append(pl.BlockSpec((_BLK, F), lambda i: (i, 0)))

    def body(*refs):
        it = iter(refs)
        xrs = [next(it) for _ in range(nx)]
        wrs = [next(it) for _ in range(nx)]
        srs = [next(it) if m is not None else None for m in modes]
        brf = next(it)
        rrf = next(it) if residual is not None else None
        orf = next(it)
        acc = None
        for xr, wr, sr, mode in zip(xrs, wrs, srs, modes):
            xv = xr[...]
            if mode == 'mean':
                xv = xv / jnp.maximum(sr[...], 1.0)
            elif mode == 'gcn':
                xv = xv / (sr[...] + 1.0)
            t = jnp.dot(xv, wr[...], preferred_element_type=jnp.float32)
            acc = t if acc is None else acc + t
        acc = acc + brf[...]
        if do_relu:
            acc = jnp.maximum(acc, 0.0)
        if rrf is not None:
            acc = rrf[...] + acc
        orf[...] = acc

    return pl.pallas_call(
        body,
        grid=(grid,),
        in_specs=in_specs,
        out_specs=pl.BlockSpec((_BLK, F), lambda i: (i, 0)),
        out_shape=jax.ShapeDtypeStruct((n, F), jnp.float32),
    )(*args)


def kernel(cell_feat, gene_idx, edge_c2g, w_c2g, edge_g2c, w_g2c, edge_g2g,
           w_g2g, params):
    p = params

    rel_c2g = _prep_rel(edge_c2g, w_c2g)
    rel_g2c = _prep_rel(edge_g2c, w_g2c)
    rel_g2g = _prep_rel(edge_g2g, w_g2g)

    deg = {'c2g': _degree(rel_c2g),
           'g2c': _degree(rel_g2c),
           'g2g': _degree(rel_g2g)}
    rels = {'c2g': rel_c2g, 'g2c': rel_g2c, 'g2g': rel_g2g}

    def conv_mean(cp, h_cell, h_gene, res_cell, res_gene):
        s1 = _seg_sum(h_cell, rels['c2g'])
        s2 = _seg_sum(h_gene, rels['g2g'])
        out_gene = _fused_linear(
            [h_gene, s1, h_gene, s2],
            [cp['c2g']['W_self'], cp['c2g']['W_neigh'],
             cp['g2g']['W_self'], cp['g2g']['W_neigh']],
            [None, (deg['c2g'], 'mean'), None, (deg['g2g'], 'mean')],
            cp['c2g']['b'] + cp['g2g']['b'], res_gene, True)
        s3 = _seg_sum(h_gene, rels['g2c'])
        out_cell = _fused_linear(
            [h_cell, s3],
            [cp['g2c']['W_self'], cp['g2c']['W_neigh']],
            [None, (deg['g2c'], 'mean')],
            cp['g2c']['b'], res_cell, True)
        return out_cell, out_gene

    def conv_gcn(cp, h_cell, h_gene, res_cell, res_gene):
        s1 = _seg_sum(h_cell, rels['c2g'])
        s2 = _seg_sum(h_gene, rels['g2g'])
        out_gene = _fused_linear(
            [s1, h_gene, s2, h_gene],
            [cp['c2g']['W_neigh'], cp['c2g']['W_neigh'],
             cp['g2g']['W_neigh'], cp['g2g']['W_neigh']],
            [(deg['c2g'], 'gcn'), (deg['c2g'], 'gcn'),
             (deg['g2g'], 'gcn'), (deg['g2g'], 'gcn')],
            cp['c2g']['b'] + cp['g2g']['b'], res_gene, True)
        s3 = _seg_sum(h_gene, rels['g2c'])
        out_cell = _fused_linear(
            [s3, h_cell],
            [cp['g2c']['W_neigh'], cp['g2c']['W_neigh']],
            [(deg['g2c'], 'gcn'), (deg['g2c'], 'gcn')],
            cp['g2c']['b'], res_cell, True)
        return out_cell, out_gene

    def conv_pool(cp, h_cell, h_gene, res_cell, res_gene):
        pre_c2g = _fused_linear([h_cell], [cp['c2g']['W_pool']], [None],
                                cp['c2g']['b_pool'], None, True)
        pre_g2g = _fused_linear([h_gene], [cp['g2g']['W_pool']], [None],
                                cp['g2g']['b_pool'], None, True)
        pre_g2c = _fused_linear([h_gene], [cp['g2c']['W_pool']], [None],
                                cp['g2c']['b_pool'], None, True)
        m1 = _seg_max(pre_c2g, rels['c2g'])
        m2 = _seg_max(pre_g2g, rels['g2g'])
        out_gene = _fused_linear(
            [h_gene, m1, h_gene, m2],
            [cp['c2g']['W_self'], cp['c2g']['W_neigh'],
             cp['g2g']['W_self'], cp['g2g']['W_neigh']],
            [None, None, None, None],
            cp['c2g']['b'] + cp['g2g']['b'], res_gene, True)
        m3 = _seg_max(pre_g2c, rels['g2c'])
        out_cell = _fused_linear(
            [h_cell, m3],
            [cp['g2c']['W_self'], cp['g2c']['W_neigh']],
            [None, None],
            cp['g2c']['b'], res_cell, True)
        return out_cell, out_gene

    gidx = jnp.pad(gene_idx, (0, GPAD - N_GENE)).astype(jnp.int32)
    h_gene = _gather_k(p['embed'], gidx)[:N_GENE]
    h_cell = cell_feat

    h_cell, h_gene = conv_mean(p['conv1'], h_cell, h_gene, None, None)
    h_cell, h_gene = conv_gcn(p['conv2'], h_cell, h_gene, h_cell, h_gene)
    h_cell, h_gene = conv_pool(p['conv3'], h_cell, h_gene, h_cell, h_gene)
    h_cell = _fused_linear([h_cell], [p['lin1']['cell']['W']], [None],
                           p['lin1']['cell']['b'], None, False)
    h_gene = _fused_linear([h_gene], [p['lin1']['gene']['W']], [None],
                           p['lin1']['gene']['b'], None, False)
    h_cell, h_gene = conv_mean(p['conv4'], h_cell, h_gene, h_cell, h_gene)
    h_cell, h_gene = conv_gcn(p['conv5'], h_cell, h_gene, h_cell, h_gene)
    h_cell, h_gene = conv_pool(p['conv6'], h_cell, h_gene, h_cell, h_gene)
    h_cell = _fused_linear([h_cell], [p['lin2']['cell']['W']], [None],
                           p['lin2']['cell']['b'], None, False)
    h_gene = _fused_linear([h_gene], [p['lin2']['gene']['W']], [None],
                           p['lin2']['gene']['b'], None, False)
    return jnp.concatenate([h_cell, h_gene], axis=0)


# speedup vs baseline: 2.4191x; 1.0051x over previous
import functools

import jax
import jax.numpy as jnp
from jax import lax
from jax.experimental import pallas as pl
from jax.experimental.pallas import tpu as pltpu
from jax.experimental.pallas import tpu_sc as plsc

F = 128
N_CELL = 50000
N_GENE = 50000
E = 200000

NC = 2
NS = 16
NW = NC * NS
L = 16

PASSES = 3
NBUCK = PASSES * NW
ROWS = 528
NPAD = NBUCK * ROWS
CH = 128
CHB = CH + 16
EP = 200960

_BLK = 1000


def _splat_i32(x):
    return jnp.zeros((L,), jnp.int32) + x


def _sread(ref, i):
    v = plsc.load_gather(ref, [_splat_i32(i)])
    return jnp.max(v)


def _seg_body(is_max, x_h, src_h, w_h, dst_h, starts_h, out_h,
              acc, src_b, w_b, dst_b, idxl, weff, dstl, rows, st_v, sem):
    c = lax.axis_index("c")
    s = lax.axis_index("s")
    wid = s * NC + c
    iot = lax.iota(jnp.int32, L)
    pltpu.sync_copy(starts_h, st_v)
    for p in range(PASSES):
        g = p * NW + wid

        def zero_body(r, _):
            rs = _splat_i32(r)
            for k in range(F // L):
                plsc.store_scatter(acc, [rs, iot + k * L],
                                   jnp.zeros((L,), jnp.float32))
            return 0
        lax.fori_loop(0, ROWS, zero_body, 0)

        lo = _sread(st_v, g)
        hi = _sread(st_v, g + 1)
        abase = (lo // 8) * 8
        n_chunks = (hi - abase + CH - 1) // CH

        def chunk_body(i, _):
            cb = abase + i * CH
            pltpu.sync_copy(src_h.at[pl.ds(cb, CHB)], src_b)
            pltpu.sync_copy(w_h.at[pl.ds(cb, CHB)], w_b)
            pltpu.sync_copy(dst_h.at[pl.ds(cb, CHB)], dst_b)
            for m in range(CH // L):
                sl = pl.ds(m * L, L)
                eidx = cb + m * L + iot
                valid = (eidx >= lo) & (eidx < hi)
                idxl[sl] = jnp.where(valid, src_b[sl], 0)
                weff[sl] = jnp.where(valid, w_b[sl], jnp.float32(0.0))
                dstl[sl] = jnp.clip(dst_b[sl] - g * ROWS, 0, ROWS - 1)
            pltpu.async_copy(x_h.at[idxl], rows, sem).wait()

            if is_max:
                U = 8

                def edge_body(eg, _):
                    base = eg * U
                    es = [_splat_i32(base + j) for j in range(U)]
                    ws = [plsc.load_gather(weff, [e]) for e in es]
                    dsv = [plsc.load_gather(dstl, [e]) for e in es]
                    for k in range(F // L):
                        col = iot + k * L
                        for j in range(U):
                            rv = plsc.load_gather(rows, [es[j], col]) * ws[j]
                            cur = plsc.load_gather(acc, [dsv[j], col])
                            plsc.store_scatter(acc, [dsv[j], col],
                                               jnp.maximum(cur, rv))
                    return 0
                lax.fori_loop(0, CH // U, edge_body, 0)
            else:
                @plsc.parallel_loop(0, CH, step=1, unroll=8)
                def _(e):
                    es = _splat_i32(e)
                    ws = plsc.load_gather(weff, [es])
                    dsv = plsc.load_gather(dstl, [es])
                    for k in range(F // L):
                        col = iot + k * L
                        rv = plsc.load_gather(rows, [es, col]) * ws
                        plsc.addupdate_scatter(acc, [dsv, col], rv)
            return 0
        lax.fori_loop(0, n_chunks, chunk_body, 0)
        pltpu.sync_copy(acc, out_h.at[pl.ds(g * ROWS, ROWS)])


def _make_seg(is_max):
    mesh = plsc.VectorSubcoreMesh(core_axis_name="c", subcore_axis_name="s",
                                  num_cores=NC, num_subcores=NS)
    return pl.kernel(
        functools.partial(_seg_body, is_max),
        out_type=jax.ShapeDtypeStruct((NPAD, F), jnp.float32),
        mesh=mesh,
        compiler_params=pltpu.CompilerParams(needs_layout_passes=False),
        scratch_types=[
            pltpu.VMEM((ROWS, F), jnp.float32),
            pltpu.VMEM((CHB,), jnp.int32),
            pltpu.VMEM((CHB,), jnp.float32),
            pltpu.VMEM((CHB,), jnp.int32),
            pltpu.VMEM((CH,), jnp.int32),
            pltpu.VMEM((CH,), jnp.float32),
            pltpu.VMEM((CH,), jnp.int32),
            pltpu.VMEM((CH, F), jnp.float32),
            pltpu.VMEM((104,), jnp.int32),
            pltpu.SemaphoreType.DMA,
        ],
    )


def _deg_body(dst_h, starts_h, out_h, dacc, dst_b, dval, dstl, st_v):
    c = lax.axis_index("c")
    s = lax.axis_index("s")
    wid = s * NC + c
    iot = lax.iota(jnp.int32, L)
    lane0 = iot == 0
    pltpu.sync_copy(starts_h, st_v)
    for p in range(PASSES):
        g = p * NW + wid

        for r in range(8):
            for k in range(F // L):
                plsc.store_scatter(dacc, [_splat_i32(r), iot + k * L],
                                   jnp.zeros((L,), jnp.float32))

        lo = _sread(st_v, g)
        hi = _sread(st_v, g + 1)
        abase = (lo // 8) * 8
        n_chunks = (hi - abase + CH - 1) // CH

        def chunk_body(i, _):
            cb = abase + i * CH
            pltpu.sync_copy(dst_h.at[pl.ds(cb, CHB)], dst_b)
            for m in range(CH // L):
                sl = pl.ds(m * L, L)
                eidx = cb + m * L + iot
                valid = (eidx >= lo) & (eidx < hi)
                dval[sl] = jnp.where(valid, jnp.float32(1.0), jnp.float32(0.0))
                dstl[sl] = jnp.clip(dst_b[sl] - g * ROWS, 0, ROWS - 1)

            @plsc.parallel_loop(0, CH, step=1, unroll=8)
            def _(e):
                es = _splat_i32(e)
                vspl = plsc.load_gather(dval, [es])
                dspl = plsc.load_gather(dstl, [es])
                plsc.addupdate_scatter(dacc, [jnp.right_shift(dspl, 7),
                                              dspl & 127], vspl, mask=lane0)
            return 0
        lax.fori_loop(0, n_chunks, chunk_body, 0)
        pltpu.sync_copy(dacc, out_h.at[g])


def _make_deg():
    mesh = plsc.VectorSubcoreMesh(core_axis_name="c", subcore_axis_name="s",
                                  num_cores=NC, num_subcores=NS)
    return pl.kernel(
        _deg_body,
        out_type=jax.ShapeDtypeStruct((NBUCK, 8, F), jnp.float32),
        mesh=mesh,
        compiler_params=pltpu.CompilerParams(needs_layout_passes=False),
        scratch_types=[
            pltpu.VMEM((8, F), jnp.float32),
            pltpu.VMEM((CHB,), jnp.int32),
            pltpu.VMEM((CH,), jnp.float32),
            pltpu.VMEM((CH,), jnp.int32),
            pltpu.VMEM((104,), jnp.int32),
        ],
    )


GR = 1568
GCH = 224
GPAD = NW * GR


def _gather_body(tab_h, idx_h, out_h, idx_b, rows, sem):
    c = lax.axis_index("c")
    s = lax.axis_index("s")
    wid = s * NC + c
    base = wid * GR
    for i in range(GR // GCH):
        off = base + i * GCH
        pltpu.sync_copy(idx_h.at[pl.ds(off, GCH)], idx_b)
        pltpu.async_copy(tab_h.at[idx_b], rows, sem).wait()
        pltpu.sync_copy(rows, out_h.at[pl.ds(off, GCH)])


def _make_gather():
    mesh = plsc.VectorSubcoreMesh(core_axis_name="c", subcore_axis_name="s",
                                  num_cores=NC, num_subcores=NS)
    return pl.kernel(
        _gather_body,
        out_type=jax.ShapeDtypeStruct((GPAD, F), jnp.float32),
        mesh=mesh,
        compiler_params=pltpu.CompilerParams(needs_layout_passes=False),
        scratch_types=[
            pltpu.VMEM((GCH,), jnp.int32),
            pltpu.VMEM((GCH, F), jnp.float32),
            pltpu.SemaphoreType.DMA,
        ],
    )


_seg_sum_k = _make_seg(False)
_seg_max_k = _make_seg(True)
_deg_k = _make_deg()
_gather_k = _make_gather()


def _prep_rel(edge, w):
    src, dst = edge[0], edge[1]
    order = jnp.argsort(dst)
    ssrc = jnp.pad(src[order], (0, EP - E))
    sw = jnp.pad(w[order], (0, EP - E))
    sdst = jnp.pad(dst[order], (0, EP - E))
    bounds = jnp.arange(NBUCK + 1, dtype=jnp.int32) * ROWS
    starts = jnp.searchsorted(dst[order], bounds, side='left').astype(jnp.int32)
    starts = jnp.pad(starts, (0, 104 - (NBUCK + 1)))
    return ssrc, sw, sdst, starts


def _seg_sum(x, rel):
    ssrc, sw, sdst, starts = rel
    return _seg_sum_k(x, ssrc, sw, sdst, starts)[:N_GENE]


def _seg_max(x, rel):
    ssrc, sw, sdst, starts = rel
    return _seg_max_k(x, ssrc, sw, sdst, starts)[:N_GENE]


def _degree(rel):
    _, _, sdst, starts = rel
    d = _deg_k(sdst, starts).reshape(NBUCK, 8 * F)[:, :ROWS]
    return d.reshape(NPAD)[:N_GENE]


def _fused_linear(xs, ws, scales, bias, residual, do_relu):
    n = xs[0].shape[0]
    nx = len(xs)
    grid = n // _BLK
    modes = [None if sc is None else sc[1] for sc in scales]
    args = []
    in_specs = []
    for x in xs:
        args.append(x)
        in_specs.append(pl.BlockSpec((_BLK, F), lambda i: (i, 0)))
    for w in ws:
        args.append(w)
        in_specs.append(pl.BlockSpec((F, F), lambda i: (0, 0)))
    for sc in scales:
        if sc is not None:
            args.append(sc[0].reshape(n, 1))
            in_specs.append(pl.BlockSpec((_BLK, 1), lambda i: (i, 0)))
    args.append(bias.reshape(1, F))
    in_specs.append(pl.BlockSpec((1, F), lambda i: (0, 0)))
    if residual is not None:
        args.append(residual)
        in_specs.append(pl.BlockSpec((_BLK, F), lambda i: (i, 0)))

    def body(*refs):
        it = iter(refs)
        xrs = [next(it) for _ in range(nx)]
        wrs = [next(it) for _ in range(nx)]
        srs = [next(it) if m is not None else None for m in modes]
        brf = next(it)
        rrf = next(it) if residual is not None else None
        orf = next(it)
        acc = None
        for xr, wr, sr, mode in zip(xrs, wrs, srs, modes):
            xv = xr[...]
            if mode == 'mean':
                xv = xv / jnp.maximum(sr[...], 1.0)
            elif mode == 'gcn':
                xv = xv / (sr[...] + 1.0)
            t = jnp.dot(xv, wr[...], preferred_element_type=jnp.float32)
            acc = t if acc is None else acc + t
        acc = acc + brf[...]
        if do_relu:
            acc = jnp.maximum(acc, 0.0)
        if rrf is not None:
            acc = rrf[...] + acc
        orf[...] = acc

    return pl.pallas_call(
        body,
        grid=(grid,),
        in_specs=in_specs,
        out_specs=pl.BlockSpec((_BLK, F), lambda i: (i, 0)),
        out_shape=jax.ShapeDtypeStruct((n, F), jnp.float32),
    )(*args)


def kernel(cell_feat, gene_idx, edge_c2g, w_c2g, edge_g2c, w_g2c, edge_g2g,
           w_g2g, params):
    p = params

    rel_c2g = _prep_rel(edge_c2g, w_c2g)
    rel_g2c = _prep_rel(edge_g2c, w_g2c)
    rel_g2g = _prep_rel(edge_g2g, w_g2g)

    deg = {'c2g': _degree(rel_c2g),
           'g2c': _degree(rel_g2c),
           'g2g': _degree(rel_g2g)}
    rels = {'c2g': rel_c2g, 'g2c': rel_g2c, 'g2g': rel_g2g}

    def conv_mean(cp, h_cell, h_gene, res_cell, res_gene):
        s1 = _seg_sum(h_cell, rels['c2g'])
        s2 = _seg_sum(h_gene, rels['g2g'])
        out_gene = _fused_linear(
            [h_gene, s1, h_gene, s2],
            [cp['c2g']['W_self'], cp['c2g']['W_neigh'],
             cp['g2g']['W_self'], cp['g2g']['W_neigh']],
            [None, (deg['c2g'], 'mean'), None, (deg['g2g'], 'mean')],
            cp['c2g']['b'] + cp['g2g']['b'], res_gene, True)
        s3 = _seg_sum(h_gene, rels['g2c'])
        out_cell = _fused_linear(
            [h_cell, s3],
            [cp['g2c']['W_self'], cp['g2c']['W_neigh']],
            [None, (deg['g2c'], 'mean')],
            cp['g2c']['b'], res_cell, True)
        return out_cell, out_gene

    def conv_gcn(cp, h_cell, h_gene, res_cell, res_gene):
        s1 = _seg_sum(h_cell, rels['c2g'])
        s2 = _seg_sum(h_gene, rels['g2g'])
        out_gene = _fused_linear(
            [s1, h_gene, s2, h_gene],
            [cp['c2g']['W_neigh'], cp['c2g']['W_neigh'],
             cp['g2g']['W_neigh'], cp['g2g']['W_neigh']],
            [(deg['c2g'], 'gcn'), (deg['c2g'], 'gcn'),
             (deg['g2g'], 'gcn'), (deg['g2g'], 'gcn')],
            cp['c2g']['b'] + cp['g2g']['b'], res_gene, True)
        s3 = _seg_sum(h_gene, rels['g2c'])
        out_cell = _fused_linear(
            [s3, h_cell],
            [cp['g2c']['W_neigh'], cp['g2c']['W_neigh']],
            [(deg['g2c'], 'gcn'), (deg['g2c'], 'gcn')],
            cp['g2c']['b'], res_cell, True)
        return out_cell, out_gene

    def conv_pool(cp, h_cell, h_gene, res_cell, res_gene):
        pre_c2g = _fused_linear([h_cell], [cp['c2g']['W_pool']], [None],
                                cp['c2g']['b_pool'], None, True)
        pre_g2g = _fused_linear([h_gene], [cp['g2g']['W_pool']], [None],
                                cp['g2g']['b_pool'], None, True)
        pre_g2c = _fused_linear([h_gene], [cp['g2c']['W_pool']], [None],
                                cp['g2c']['b_pool'], None, True)
        m1 = _seg_max(pre_c2g, rels['c2g'])
        m2 = _seg_max(pre_g2g, rels['g2g'])
        out_gene = _fused_linear(
            [h_gene, m1, h_gene, m2],
            [cp['c2g']['W_self'], cp['c2g']['W_neigh'],
             cp['g2g']['W_self'], cp['g2g']['W_neigh']],
            [None, None, None, None],
            cp['c2g']['b'] + cp['g2g']['b'], res_gene, True)
        m3 = _seg_max(pre_g2c, rels['g2c'])
        out_cell = _fused_linear(
            [h_cell, m3],
            [cp['g2c']['W_self'], cp['g2c']['W_neigh']],
            [None, None],
            cp['g2c']['b'], res_cell, True)
        return out_cell, out_gene

    gidx = jnp.pad(gene_idx, (0, GPAD - N_GENE)).astype(jnp.int32)
    h_gene = _gather_k(p['embed'], gidx)[:N_GENE]
    h_cell = cell_feat

    h_cell, h_gene = conv_mean(p['conv1'], h_cell, h_gene, None, None)
    h_cell, h_gene = conv_gcn(p['conv2'], h_cell, h_gene, h_cell, h_gene)
    h_cell, h_gene = conv_pool(p['conv3'], h_cell, h_gene, h_cell, h_gene)
    h_cell = _fused_linear([h_cell], [p['lin1']['cell']['W']], [None],
                           p['lin1']['cell']['b'], None, False)
    h_gene = _fused_linear([h_gene], [p['lin1']['gene']['W']], [None],
                           p['lin1']['gene']['b'], None, False)
    h_cell, h_gene = conv_mean(p['conv4'], h_cell, h_gene, h_cell, h_gene)
    h_cell, h_gene = conv_gcn(p['conv5'], h_cell, h_gene, h_cell, h_gene)
    h_cell, h_gene = conv_pool(p['conv6'], h_cell, h_gene, h_cell, h_gene)
    h_cell = _fused_linear([h_cell], [p['lin2']['cell']['W']], [None],
                           p['lin2']['cell']['b'], None, False)
    h_gene = _fused_linear([h_gene], [p['lin2']['gene']['W']], [None],
                           p['lin2']['gene']['b'], None, False)
    return jnp.concatenate([h_cell, h_gene], axis=0)
